# baseline (speedup 1.0000x reference)
import functools

import jax
import jax.numpy as jnp
from jax import lax
from jax.experimental import pallas as pl
from jax.experimental.pallas import tpu as pltpu
from jax.experimental.pallas import tpu_sc as plsc

_B = 16
_P = 128
_T = 512
_SRC_LEN = 8
_PATH_LEN = 12
_IN_DIM = 128
_H_DIM = 256
_NUM_LAYERS = 2
_NW = 32


def _sc_ctx_gather(ctx_i, ctx_t):
    ctx_per_w = (_B * _T) // _NW
    mesh = plsc.VectorSubcoreMesh(core_axis_name="c", subcore_axis_name="s")

    @functools.partial(
        pl.kernel,
        out_type=jax.ShapeDtypeStruct((_B * _T, _IN_DIM), jnp.float32),
        mesh=mesh,
        scratch_types=[
            pltpu.VMEM((ctx_per_w,), jnp.int32),
            pltpu.VMEM((ctx_per_w, _IN_DIM), jnp.float32),
            pltpu.SemaphoreType.DMA,
        ],
    )
    def k(ctxi_r, ctxt_r, ctx_o, idxc, rows, sem):
        wid = lax.axis_index("s") * 2 + lax.axis_index("c")
        cb = wid * ctx_per_w
        pltpu.sync_copy(ctxi_r.at[pl.ds(cb, ctx_per_w)], idxc)
        pltpu.async_copy(ctxt_r.at[idxc], rows, sem).wait()
        pltpu.sync_copy(rows, ctx_o.at[pl.ds(cb, ctx_per_w)])

    return k(ctx_i, ctx_t)


_N_ROWS = _B * _P
_OFF_STOK = 0
_OFF_TTOK = _OFF_STOK + _N_ROWS * _SRC_LEN
_OFF_PTOK = _OFF_TTOK + _N_ROWS * _SRC_LEN
_OFF_SPOS = _OFF_PTOK + _N_ROWS * _PATH_LEN
_OFF_TPOS = _OFF_SPOS + _N_ROWS * _SRC_LEN
_OFF_PPOS = _OFF_TPOS + _N_ROWS * _SRC_LEN
_IDX_TOTAL = _OFF_PPOS + _N_ROWS * _PATH_LEN


def _sc_path_sums(idx_all, st_t, path_t, pos_aug):
    rows_per_w = _N_ROWS // _NW
    mesh = plsc.VectorSubcoreMesh(core_axis_name="c", subcore_axis_name="s")

    @functools.partial(
        pl.kernel,
        out_type=jax.ShapeDtypeStruct((3 * _N_ROWS, _IN_DIM), jnp.float32),
        mesh=mesh,
        scratch_types=[
            pltpu.VMEM((16 * _PATH_LEN,), jnp.int32),
            pltpu.VMEM((16 * _PATH_LEN,), jnp.int32),
            pltpu.VMEM((16 * _PATH_LEN,), jnp.int32),
            pltpu.VMEM((16 * _PATH_LEN,), jnp.int32),
            pltpu.VMEM((2, 16 * _PATH_LEN, _IN_DIM), jnp.float32),
            pltpu.VMEM((2, 16 * _PATH_LEN, _IN_DIM), jnp.float32),
            pltpu.VMEM((2, 16, _IN_DIM), jnp.float32),
            pltpu.SemaphoreType.DMA((2,)),
            pltpu.SemaphoreType.DMA((2,)),
            pltpu.SemaphoreType.DMA((2,)),
        ],
    )
    def k(idx_r, st_r, path_r, pos_r, sum_o,
          idxt0, idxt1, idxp0, idxp1, rows, prows, acc, sem_t, sem_p, sem_o):
        idxt_bufs = (idxt0, idxt1)
        idxp_bufs = (idxp0, idxp1)
        wid = lax.axis_index("s") * 2 + lax.axis_index("c")
        nch = rows_per_w // 16

        def reduce_rep(tok_off, pos_off, table, out_base, G):
            base = wid * rows_per_w

            def start(c):
                b = c % 2
                rb = base + c * 16
                it = idxt_bufs[b].at[pl.ds(0, 16 * G)]
                ip = idxp_bufs[b].at[pl.ds(0, 16 * G)]
                pltpu.sync_copy(idx_r.at[pl.ds(tok_off + rb * G, 16 * G)], it)
                pltpu.sync_copy(idx_r.at[pl.ds(pos_off + rb * G, 16 * G)], ip)
                d1 = pltpu.async_copy(table.at[it],
                                      rows.at[b, pl.ds(0, 16 * G)], sem_t.at[b])
                d2 = pltpu.async_copy(pos_r.at[ip],
                                      prows.at[b, pl.ds(0, 16 * G)], sem_p.at[b])
                return d1, d2

            descs = {0: start(0)}
            out_descs = {}
            for c in range(nch):
                b = c % 2
                if c + 1 < nch:
                    descs[c + 1] = start(c + 1)
                d1, d2 = descs[c]
                d1.wait()
                d2.wait()
                if c >= 2:
                    out_descs[c - 2].wait()

                def rowfn(r, carry2):
                    for v in range(_IN_DIM // 16):
                        s = rows[b, r * G, pl.ds(v * 16, 16)]
                        for j in range(1, G):
                            s = s + rows[b, r * G + j, pl.ds(v * 16, 16)]
                        for j in range(G):
                            s = s + prows[b, r * G + j, pl.ds(v * 16, 16)]
                        acc[b, r, pl.ds(v * 16, 16)] = s
                    return carry2

                lax.fori_loop(0, 16, rowfn, 0)
                rb = base + c * 16
                out_descs[c] = pltpu.async_copy(
                    acc.at[b], sum_o.at[pl.ds(out_base + rb, 16)], sem_o.at[b])
            out_descs[nch - 2].wait()
            out_descs[nch - 1].wait()

        reduce_rep(_OFF_STOK, _OFF_SPOS, st_r, 0, _SRC_LEN)
        reduce_rep(_OFF_TTOK, _OFF_TPOS, st_r, _N_ROWS, _SRC_LEN)
        reduce_rep(_OFF_PTOK, _OFF_PPOS, path_r, 2 * _N_ROWS, _PATH_LEN)

    return k(idx_all, st_t, path_t, pos_aug)


def _fuse_body(sums, stok, ttok, ptok, r0st, r0p, wp, bp, wm, bm,
               csum, dlens, mixed_o, h_o):
    n = _B * _P
    sums_v = sums[...]

    def rep(sum_v, tok_ref, row0_ref, G):
        tok = tok_ref[...]
        mask = (tok != 0).astype(jnp.float32)
        cnt = mask.sum(axis=1, keepdims=True)
        cnt0 = G - cnt
        return (sum_v - cnt0 * row0_ref[0:1, :]) / jnp.maximum(cnt, 1.0)

    sr = rep(sums_v[0:n], stok, r0st, _SRC_LEN)
    tr = rep(sums_v[n:2 * n], ttok, r0st, _SRC_LEN)
    pr = rep(sums_v[2 * n:], ptok, r0p, _PATH_LEN)

    wp_v = wp[...].astype(jnp.bfloat16)
    ep = jnp.tanh(
        jnp.dot(sr.astype(jnp.bfloat16), wp_v[0:128], preferred_element_type=jnp.float32)
        + jnp.dot(tr.astype(jnp.bfloat16), wp_v[128:256], preferred_element_type=jnp.float32)
        + jnp.dot(pr.astype(jnp.bfloat16), wp_v[256:384], preferred_element_type=jnp.float32)
        + bp[...])
    mixed_o[...] = jnp.tanh(
        jnp.dot(ep.astype(jnp.bfloat16), wm[...].astype(jnp.bfloat16),
                preferred_element_type=jnp.float32) + bm[...])
    gids = lax.broadcasted_iota(jnp.int32, (_B, n), 1) // _P
    sel = (gids == lax.broadcasted_iota(jnp.int32, (_B, n), 0)).astype(jnp.float32)
    psum = jnp.dot(sel, ep, preferred_element_type=jnp.float32)
    hf = (psum + csum[...]) / dlens[...]
    h_o[...] = jnp.broadcast_to(hf[None], (_NUM_LAYERS, _B, _H_DIM))


def _tc_fuse(sums, stok, ttok, ptok, st_emb, path_emb, wp, bp, wm, bm,
             csum, dlens):
    n = _B * _P
    full = lambda shape: pl.BlockSpec(shape, lambda i: tuple(0 for _ in shape))
    return pl.pallas_call(
        _fuse_body,
        grid=(1,),
        in_specs=[
            full((3 * n, _IN_DIM)),
            full((n, _SRC_LEN)),
            full((n, _SRC_LEN)),
            full((n, _PATH_LEN)),
            pl.BlockSpec((8, _IN_DIM), lambda i: (0, 0)),
            pl.BlockSpec((8, _IN_DIM), lambda i: (0, 0)),
            full((3 * _IN_DIM, _H_DIM)),
            full((1, _H_DIM)),
            full((_H_DIM, _H_DIM)),
            full((1, _H_DIM)),
            full((_B, _H_DIM)),
            full((_B, 1)),
        ],
        out_specs=[
            full((n, _H_DIM)),
            full((_NUM_LAYERS, _B, _H_DIM)),
        ],
        out_shape=(
            jax.ShapeDtypeStruct((n, _H_DIM), jnp.float32),
            jax.ShapeDtypeStruct((_NUM_LAYERS, _B, _H_DIM), jnp.float32),
        ),
    )(sums, stok, ttok, ptok, st_emb, path_emb, wp, bp, wm, bm, csum, dlens)


_TCHUNK = 64


def _gru_body(ctx_ref, wx_ref, wh_ref, bg_ref, len_ref,
              enc_ref, csum_o, h_s, csum_s, xg_s, ys_s):
    i = pl.program_id(0)

    @pl.when(i == 0)
    def _init():
        h_s[...] = jnp.zeros_like(h_s)
        csum_s[...] = jnp.zeros_like(csum_s)

    x = ctx_ref[...].astype(jnp.bfloat16)
    xg_s[...] = jnp.dot(x, wx_ref[...].astype(jnp.bfloat16),
                        preferred_element_type=jnp.float32) + bg_ref[...]
    lens = len_ref[...]
    wh = wh_ref[...].astype(jnp.bfloat16)
    _G = _B // 2
    def hdot(h):
        return jnp.dot(h.astype(jnp.bfloat16), wh, preferred_element_type=jnp.float32)

    sig = jax.nn.sigmoid

    def gates(xgt, hg, h, t, lens_g):
        xr = xgt[:, 0:_H_DIM]
        xz = xgt[:, _H_DIM:2 * _H_DIM]
        xn = xgt[:, 2 * _H_DIM:]
        hr = hg[:, 0:_H_DIM]
        hz = hg[:, _H_DIM:2 * _H_DIM]
        hn = hg[:, 2 * _H_DIM:]
        r = sig(xr + hr)
        z = sig(xz + hz)
        nn_ = jnp.tanh(xn + r * hn)
        h_new = (1.0 - z) * nn_ + z * h
        m = (i * _TCHUNK + t) < lens_g
        h_out = jnp.where(m, h_new, h)
        return h_out, m

    def step(t, carry):
        h_a, h_b, hg_b = carry
        hg_a = hdot(h_a)
        xgt_b = xg_s[pl.ds(t * _B + _G, _G), :]
        hb_out, mb = gates(xgt_b, hg_b, h_b, t, lens[_G:])
        ys_s[pl.ds(t * _B + _G, _G), :] = hb_out
        csum_s[_G:, :] = csum_s[_G:, :] + jnp.where(mb, hb_out, 0.0)
        hg_b_next = hdot(hb_out)
        xgt_a = xg_s[pl.ds(t * _B, _G), :]
        ha_out, ma = gates(xgt_a, hg_a, h_a, t, lens[0:_G])
        ys_s[pl.ds(t * _B, _G), :] = ha_out
        csum_s[0:_G, :] = csum_s[0:_G, :] + jnp.where(ma, ha_out, 0.0)
        return ha_out, hb_out, hg_b_next

    _UNROLL = 16

    def stepu(u, carry):
        for k in range(_UNROLL):
            carry = step(_UNROLL * u + k, carry)
        return carry

    h0 = h_s[...]
    hg_b0 = hdot(h0[_G:])
    ha_f, hb_f, _ = lax.fori_loop(0, _TCHUNK // _UNROLL, stepu,
                                  (h0[0:_G], h0[_G:], hg_b0))
    h_s[...] = jnp.concatenate([ha_f, hb_f], axis=0)
    enc_ref[...] = ys_s[...].reshape(_TCHUNK, _B, _H_DIM).transpose(1, 0, 2)

    @pl.when(i == (_T // _TCHUNK) - 1)
    def _fin():
        csum_o[...] = csum_s[...]


def _tc_gru(ctx_in_tm, wx, wh, bg, lens):
    nchunks = _T // _TCHUNK
    return pl.pallas_call(
        _gru_body,
        grid=(nchunks,),
        in_specs=[
            pl.BlockSpec((_TCHUNK * _B, _IN_DIM), lambda i: (i, 0)),
            pl.BlockSpec((_IN_DIM, 3 * _H_DIM), lambda i: (0, 0)),
            pl.BlockSpec((_H_DIM, 3 * _H_DIM), lambda i: (0, 0)),
            pl.BlockSpec((1, 3 * _H_DIM), lambda i: (0, 0)),
            pl.BlockSpec((_B, 1), lambda i: (0, 0)),
        ],
        out_specs=[
            pl.BlockSpec((_B, _TCHUNK, _H_DIM), lambda i: (0, i, 0)),
            pl.BlockSpec((_B, _H_DIM), lambda i: (0, 0)),
        ],
        out_shape=(
            jax.ShapeDtypeStruct((_B, _T, _H_DIM), jnp.float32),
            jax.ShapeDtypeStruct((_B, _H_DIM), jnp.float32),
        ),
        scratch_shapes=[
            pltpu.VMEM((_B, _H_DIM), jnp.float32),
            pltpu.VMEM((_B, _H_DIM), jnp.float32),
            pltpu.VMEM((_TCHUNK * _B, 3 * _H_DIM), jnp.float32),
            pltpu.VMEM((_TCHUNK * _B, _H_DIM), jnp.float32),
        ],
    )(ctx_in_tm, wx, wh, bg, lens)


def kernel(packed_srcs, packed_srcs_positions, packed_tgts, packed_tgts_positions,
           packed_paths, packed_paths_positions, packed_ctx, ctx_lengths,
           focus_num_of_paths, path_emb, src_tgt_emb, pos_emb, ctx_table,
           W_path, b_path, W_mix, b_mix, Wx, Wh, bg):
    ctx_rows_tm = _sc_ctx_gather(packed_ctx.T.reshape(-1), ctx_table)

    pos_aug = jnp.concatenate(
        [pos_emb, jnp.zeros((1, _IN_DIM), jnp.float32)], axis=0)
    idx_all = jnp.concatenate([
        packed_srcs.reshape(-1), packed_tgts.reshape(-1),
        packed_paths.reshape(-1),
        jnp.where(packed_srcs == 0, 32, packed_srcs_positions).reshape(-1),
        jnp.where(packed_tgts == 0, 32, packed_tgts_positions).reshape(-1),
        jnp.where(packed_paths == 0, 32, packed_paths_positions).reshape(-1),
    ])
    sums = _sc_path_sums(idx_all, src_tgt_emb, path_emb, pos_aug)

    lens = ctx_lengths.reshape(_B, 1)
    enc, csum = _tc_gru(ctx_rows_tm, Wx, Wh, bg.reshape(1, -1), lens)

    dlens = (focus_num_of_paths + ctx_lengths).astype(jnp.float32).reshape(_B, 1)
    mixed, h = _tc_fuse(
        sums, packed_srcs, packed_tgts, packed_paths,
        src_tgt_emb, path_emb,
        W_path, b_path.reshape(1, -1), W_mix, b_mix.reshape(1, -1),
        csum, dlens)

    return (mixed.reshape(_B, _P, _H_DIM), enc, h)


# speedup vs baseline: 1.0708x; 1.0708x over previous
import functools

import jax
import jax.numpy as jnp
from jax import lax
from jax.experimental import pallas as pl
from jax.experimental.pallas import tpu as pltpu
from jax.experimental.pallas import tpu_sc as plsc

_B = 16
_P = 128
_T = 512
_SRC_LEN = 8
_PATH_LEN = 12
_IN_DIM = 128
_H_DIM = 256
_NUM_LAYERS = 2
_NW = 32


def _sc_ctx_gather(ctx_i, ctx_t):
    ctx_per_w = (_B * _T) // _NW
    mesh = plsc.VectorSubcoreMesh(core_axis_name="c", subcore_axis_name="s")

    @functools.partial(
        pl.kernel,
        out_type=jax.ShapeDtypeStruct((_B * _T, _IN_DIM), jnp.float32),
        mesh=mesh,
        scratch_types=[
            pltpu.VMEM((ctx_per_w,), jnp.int32),
            pltpu.VMEM((ctx_per_w, _IN_DIM), jnp.float32),
            pltpu.SemaphoreType.DMA,
        ],
    )
    def k(ctxi_r, ctxt_r, ctx_o, idxc, rows, sem):
        wid = lax.axis_index("s") * 2 + lax.axis_index("c")
        cb = wid * ctx_per_w
        pltpu.sync_copy(ctxi_r.at[pl.ds(cb, ctx_per_w)], idxc)
        pltpu.async_copy(ctxt_r.at[idxc], rows, sem).wait()
        pltpu.sync_copy(rows, ctx_o.at[pl.ds(cb, ctx_per_w)])

    return k(ctx_i, ctx_t)


_N_ROWS = _B * _P
_OFF_STOK = 0
_OFF_TTOK = _OFF_STOK + _N_ROWS * _SRC_LEN
_OFF_PTOK = _OFF_TTOK + _N_ROWS * _SRC_LEN
_IDX_TOTAL = _OFF_PTOK + _N_ROWS * _PATH_LEN


def _sc_path_sums(idx_all, st_t, path_t):
    rows_per_w = _N_ROWS // _NW
    mesh = plsc.VectorSubcoreMesh(core_axis_name="c", subcore_axis_name="s")

    @functools.partial(
        pl.kernel,
        out_type=jax.ShapeDtypeStruct((3 * _N_ROWS, _IN_DIM), jnp.float32),
        mesh=mesh,
        scratch_types=[
            pltpu.VMEM((16 * _SRC_LEN,), jnp.int32),
            pltpu.VMEM((16 * _PATH_LEN,), jnp.int32),
            pltpu.VMEM((16 * _PATH_LEN, _IN_DIM), jnp.float32),
            pltpu.VMEM((16, _IN_DIM), jnp.float32),
            pltpu.SemaphoreType.DMA,
        ],
    )
    def k(idx_r, st_r, path_r, sum_o, idx8, idx12, rows, acc, sem):
        wid = lax.axis_index("s") * 2 + lax.axis_index("c")

        def reduce_rep(tok_off, table, out_base, G, idx_v):
            base = wid * rows_per_w

            def chunk(c, carry):
                rb = base + c * 16
                pltpu.sync_copy(idx_r.at[pl.ds(tok_off + rb * G, 16 * G)], idx_v)
                pltpu.async_copy(table.at[idx_v],
                                 rows.at[pl.ds(0, 16 * G)], sem).wait()

                def rowfn(r, carry2):
                    for v in range(_IN_DIM // 16):
                        s = rows[r * G, pl.ds(v * 16, 16)]
                        for j in range(1, G):
                            s = s + rows[r * G + j, pl.ds(v * 16, 16)]
                        acc[r, pl.ds(v * 16, 16)] = s
                    return carry2

                lax.fori_loop(0, 16, rowfn, 0)
                pltpu.sync_copy(acc, sum_o.at[pl.ds(out_base + rb, 16)])
                return carry

            lax.fori_loop(0, rows_per_w // 16, chunk, 0)

        reduce_rep(_OFF_STOK, st_r, 0, _SRC_LEN, idx8)
        reduce_rep(_OFF_TTOK, st_r, _N_ROWS, _SRC_LEN, idx8)
        reduce_rep(_OFF_PTOK, path_r, 2 * _N_ROWS, _PATH_LEN, idx12)

    return k(idx_all, st_t, path_t)


def _fuse_body(sums, stok, spos, ttok, tpos, ptok, ppos, r0st, r0p, pose,
               wp, bp, wm, bm, csum, dlens, mixed_o, h_o):
    n = _B * _P
    sums_v = sums[...]
    pos_tab = pose[...].astype(jnp.bfloat16)

    def rep(sum_v, tok_ref, pos_ref, row0_ref, G):
        tok = tok_ref[...]
        posi = pos_ref[...]
        mask = (tok != 0).astype(jnp.float32)
        cnt = mask.sum(axis=1, keepdims=True)
        cnt0 = G - cnt
        iota32 = lax.broadcasted_iota(jnp.int32, (n, 32), 1)
        hist = jnp.zeros((n, 32), jnp.float32)
        for j in range(G):
            hist = hist + (posi[:, j:j + 1] == iota32).astype(jnp.float32) * mask[:, j:j + 1]
        pos_contrib = jnp.dot(hist.astype(jnp.bfloat16), pos_tab,
                              preferred_element_type=jnp.float32)
        return (sum_v - cnt0 * row0_ref[0:1, :] + pos_contrib) / jnp.maximum(cnt, 1.0)

    sr = rep(sums_v[0:n], stok, spos, r0st, _SRC_LEN)
    tr = rep(sums_v[n:2 * n], ttok, tpos, r0st, _SRC_LEN)
    pr = rep(sums_v[2 * n:], ptok, ppos, r0p, _PATH_LEN)

    wp_v = wp[...].astype(jnp.bfloat16)
    ep = jnp.tanh(
        jnp.dot(sr.astype(jnp.bfloat16), wp_v[0:128], preferred_element_type=jnp.float32)
        + jnp.dot(tr.astype(jnp.bfloat16), wp_v[128:256], preferred_element_type=jnp.float32)
        + jnp.dot(pr.astype(jnp.bfloat16), wp_v[256:384], preferred_element_type=jnp.float32)
        + bp[...])
    mixed_o[...] = jnp.tanh(
        jnp.dot(ep.astype(jnp.bfloat16), wm[...].astype(jnp.bfloat16),
                preferred_element_type=jnp.float32) + bm[...])
    gids = lax.broadcasted_iota(jnp.int32, (_B, n), 1) // _P
    sel = (gids == lax.broadcasted_iota(jnp.int32, (_B, n), 0)).astype(jnp.float32)
    psum = jnp.dot(sel, ep, preferred_element_type=jnp.float32)
    hf = (psum + csum[...]) / dlens[...]
    h_o[...] = jnp.broadcast_to(hf[None], (_NUM_LAYERS, _B, _H_DIM))


def _tc_fuse(sums, stok, spos, ttok, tpos, ptok, ppos, st_emb, path_emb,
             pos_emb, wp, bp, wm, bm, csum, dlens):
    n = _B * _P
    full = lambda shape: pl.BlockSpec(shape, lambda i: tuple(0 for _ in shape))
    return pl.pallas_call(
        _fuse_body,
        grid=(1,),
        in_specs=[
            full((3 * n, _IN_DIM)),
            full((n, _SRC_LEN)),
            full((n, _SRC_LEN)),
            full((n, _SRC_LEN)),
            full((n, _SRC_LEN)),
            full((n, _PATH_LEN)),
            full((n, _PATH_LEN)),
            pl.BlockSpec((8, _IN_DIM), lambda i: (0, 0)),
            pl.BlockSpec((8, _IN_DIM), lambda i: (0, 0)),
            full((32, _IN_DIM)),
            full((3 * _IN_DIM, _H_DIM)),
            full((1, _H_DIM)),
            full((_H_DIM, _H_DIM)),
            full((1, _H_DIM)),
            full((_B, _H_DIM)),
            full((_B, 1)),
        ],
        out_specs=[
            full((n, _H_DIM)),
            full((_NUM_LAYERS, _B, _H_DIM)),
        ],
        out_shape=(
            jax.ShapeDtypeStruct((n, _H_DIM), jnp.float32),
            jax.ShapeDtypeStruct((_NUM_LAYERS, _B, _H_DIM), jnp.float32),
        ),
    )(sums, stok, spos, ttok, tpos, ptok, ppos, st_emb, path_emb, pos_emb,
      wp, bp, wm, bm, csum, dlens)


_TCHUNK = 64


def _gru_body(ctx_ref, wx_ref, wh_ref, bg_ref, len_ref,
              enc_ref, csum_o, h_s, csum_s, xg_s, ys_s):
    i = pl.program_id(0)

    @pl.when(i == 0)
    def _init():
        h_s[...] = jnp.zeros_like(h_s)
        csum_s[...] = jnp.zeros_like(csum_s)

    x = ctx_ref[...].astype(jnp.bfloat16)
    xg_s[...] = jnp.dot(x, wx_ref[...].astype(jnp.bfloat16),
                        preferred_element_type=jnp.float32) + bg_ref[...]
    lens = len_ref[...]
    wh = wh_ref[...].astype(jnp.bfloat16)
    _G = _B // 2
    def hdot(h):
        return jnp.dot(h.astype(jnp.bfloat16), wh, preferred_element_type=jnp.float32)

    sig = jax.nn.sigmoid

    def gates(xgt, hg, h, t, lens_g):
        xr = xgt[:, 0:_H_DIM]
        xz = xgt[:, _H_DIM:2 * _H_DIM]
        xn = xgt[:, 2 * _H_DIM:]
        hr = hg[:, 0:_H_DIM]
        hz = hg[:, _H_DIM:2 * _H_DIM]
        hn = hg[:, 2 * _H_DIM:]
        r = sig(xr + hr)
        z = sig(xz + hz)
        nn_ = jnp.tanh(xn + r * hn)
        h_new = (1.0 - z) * nn_ + z * h
        m = (i * _TCHUNK + t) < lens_g
        h_out = jnp.where(m, h_new, h)
        return h_out, m

    def step(t, carry):
        h_a, h_b, hg_b = carry
        hg_a = hdot(h_a)
        xgt_b = xg_s[pl.ds(t * _B + _G, _G), :]
        hb_out, mb = gates(xgt_b, hg_b, h_b, t, lens[_G:])
        ys_s[pl.ds(t * _B + _G, _G), :] = hb_out
        csum_s[_G:, :] = csum_s[_G:, :] + jnp.where(mb, hb_out, 0.0)
        hg_b_next = hdot(hb_out)
        xgt_a = xg_s[pl.ds(t * _B, _G), :]
        ha_out, ma = gates(xgt_a, hg_a, h_a, t, lens[0:_G])
        ys_s[pl.ds(t * _B, _G), :] = ha_out
        csum_s[0:_G, :] = csum_s[0:_G, :] + jnp.where(ma, ha_out, 0.0)
        return ha_out, hb_out, hg_b_next

    _UNROLL = 16

    def stepu(u, carry):
        for k in range(_UNROLL):
            carry = step(_UNROLL * u + k, carry)
        return carry

    h0 = h_s[...]
    hg_b0 = hdot(h0[_G:])
    ha_f, hb_f, _ = lax.fori_loop(0, _TCHUNK // _UNROLL, stepu,
                                  (h0[0:_G], h0[_G:], hg_b0))
    h_s[...] = jnp.concatenate([ha_f, hb_f], axis=0)
    enc_ref[...] = ys_s[...].reshape(_TCHUNK, _B, _H_DIM).transpose(1, 0, 2)

    @pl.when(i == (_T // _TCHUNK) - 1)
    def _fin():
        csum_o[...] = csum_s[...]


def _tc_gru(ctx_in_tm, wx, wh, bg, lens):
    nchunks = _T // _TCHUNK
    return pl.pallas_call(
        _gru_body,
        grid=(nchunks,),
        in_specs=[
            pl.BlockSpec((_TCHUNK * _B, _IN_DIM), lambda i: (i, 0)),
            pl.BlockSpec((_IN_DIM, 3 * _H_DIM), lambda i: (0, 0)),
            pl.BlockSpec((_H_DIM, 3 * _H_DIM), lambda i: (0, 0)),
            pl.BlockSpec((1, 3 * _H_DIM), lambda i: (0, 0)),
            pl.BlockSpec((_B, 1), lambda i: (0, 0)),
        ],
        out_specs=[
            pl.BlockSpec((_B, _TCHUNK, _H_DIM), lambda i: (0, i, 0)),
            pl.BlockSpec((_B, _H_DIM), lambda i: (0, 0)),
        ],
        out_shape=(
            jax.ShapeDtypeStruct((_B, _T, _H_DIM), jnp.float32),
            jax.ShapeDtypeStruct((_B, _H_DIM), jnp.float32),
        ),
        scratch_shapes=[
            pltpu.VMEM((_B, _H_DIM), jnp.float32),
            pltpu.VMEM((_B, _H_DIM), jnp.float32),
            pltpu.VMEM((_TCHUNK * _B, 3 * _H_DIM), jnp.float32),
            pltpu.VMEM((_TCHUNK * _B, _H_DIM), jnp.float32),
        ],
    )(ctx_in_tm, wx, wh, bg, lens)


def kernel(packed_srcs, packed_srcs_positions, packed_tgts, packed_tgts_positions,
           packed_paths, packed_paths_positions, packed_ctx, ctx_lengths,
           focus_num_of_paths, path_emb, src_tgt_emb, pos_emb, ctx_table,
           W_path, b_path, W_mix, b_mix, Wx, Wh, bg):
    ctx_rows_tm = _sc_ctx_gather(packed_ctx.T.reshape(-1), ctx_table)

    idx_all = jnp.concatenate([
        packed_srcs.reshape(-1), packed_tgts.reshape(-1),
        packed_paths.reshape(-1),
    ])
    sums = _sc_path_sums(idx_all, src_tgt_emb, path_emb)

    lens = ctx_lengths.reshape(_B, 1)
    enc, csum = _tc_gru(ctx_rows_tm, Wx, Wh, bg.reshape(1, -1), lens)

    dlens = (focus_num_of_paths + ctx_lengths).astype(jnp.float32).reshape(_B, 1)
    mixed, h = _tc_fuse(
        sums, packed_srcs, packed_srcs_positions,
        packed_tgts, packed_tgts_positions,
        packed_paths, packed_paths_positions,
        src_tgt_emb, path_emb, pos_emb,
        W_path, b_path.reshape(1, -1), W_mix, b_mix.reshape(1, -1),
        csum, dlens)

    return (mixed.reshape(_B, _P, _H_DIM), enc, h)
